# baseline (speedup 1.0000x reference)
import functools

import jax
import jax.numpy as jnp
from jax import lax
from jax.experimental import pallas as pl
from jax.experimental.pallas import tpu as pltpu
from jax.experimental.pallas import tpu_sc as plsc

D_MODEL = 1024
D_FF = 4096
N_EXP = 8
N_TOKENS = 4096
GATE_TILE = 1024
T = 512
MAX_TILES = N_TOKENS // T + N_EXP
PAD_ROWS = MAX_TILES * T

NW = 32
TOK_PER_W = N_TOKENS // NW
L = 16
CH = TOK_PER_W // L


def _gate_body(x_ref, gw_ref, gb_ref, top1_ref, pos_ref, cnt_ref, loss_ref,
               carry_ref, imp_ref):
    i = pl.program_id(0)

    @pl.when(i == 0)
    def _():
        carry_ref[...] = jnp.zeros_like(carry_ref)
        imp_ref[...] = jnp.zeros_like(imp_ref)

    x = x_ref[...]
    logits = lax.dot_general(x, gw_ref[...], (((1,), (1,)), ((), ())),
                             preferred_element_type=jnp.float32)
    logits = logits + gb_ref[...]
    m = jnp.max(logits, axis=-1, keepdims=True)
    e = jnp.exp(logits - m)
    probs = e / jnp.sum(e, axis=-1, keepdims=True)
    top1 = jnp.argmax(probs, axis=-1).astype(jnp.int32)

    lanes = lax.broadcasted_iota(jnp.int32, (GATE_TILE, 128), 1)
    oh = (lanes == top1[:, None]).astype(jnp.float32)

    r = lax.broadcasted_iota(jnp.int32, (GATE_TILE, GATE_TILE), 0)
    c = lax.broadcasted_iota(jnp.int32, (GATE_TILE, GATE_TILE), 1)
    tri = (c < r).astype(jnp.bfloat16)
    excl = lax.dot_general(tri, oh.astype(jnp.bfloat16),
                           (((1,), (0,)), ((), ())),
                           preferred_element_type=jnp.float32)
    pos = jnp.sum((excl + carry_ref[...]) * oh, axis=1)

    top1_ref[...] = top1[:, None]
    pos_ref[...] = pos.astype(jnp.int32)[:, None]

    imp_ref[...] += jnp.sum(probs, axis=0, keepdims=True)
    new_carry = carry_ref[...] + jnp.sum(oh, axis=0, keepdims=True)
    carry_ref[...] = new_carry
    cnt_ref[...] = new_carry.astype(jnp.int32)
    imp_mean = imp_ref[...] / N_TOKENS
    load = new_carry[:, :N_EXP] / N_TOKENS
    loss_ref[...] = jnp.sum(N_EXP * imp_mean * load).reshape(1, 1)


_gate_call = pl.pallas_call(
    _gate_body,
    grid=(N_TOKENS // GATE_TILE,),
    in_specs=[
        pl.BlockSpec((GATE_TILE, D_MODEL), lambda i: (i, 0)),
        pl.BlockSpec((N_EXP, D_MODEL), lambda i: (0, 0)),
        pl.BlockSpec((1, N_EXP), lambda i: (0, 0)),
    ],
    out_specs=[
        pl.BlockSpec((GATE_TILE, 1), lambda i: (i, 0)),
        pl.BlockSpec((GATE_TILE, 1), lambda i: (i, 0)),
        pl.BlockSpec((1, 128), lambda i: (0, 0)),
        pl.BlockSpec((1, 1), lambda i: (0, 0)),
    ],
    out_shape=[
        jax.ShapeDtypeStruct((N_TOKENS, 1), jnp.int32),
        jax.ShapeDtypeStruct((N_TOKENS, 1), jnp.int32),
        jax.ShapeDtypeStruct((1, 128), jnp.int32),
        jax.ShapeDtypeStruct((1, 1), jnp.float32),
    ],
    scratch_shapes=[
        pltpu.VMEM((1, 128), jnp.float32),
        pltpu.VMEM((1, N_EXP), jnp.float32),
    ],
    compiler_params=pltpu.CompilerParams(
        dimension_semantics=("arbitrary",)),
)


@functools.lru_cache(maxsize=None)
def _sc_kernels():
    mesh = plsc.VectorSubcoreMesh(core_axis_name="c", subcore_axis_name="s")

    @functools.partial(
        pl.kernel,
        mesh=mesh,
        out_type=[
            jax.ShapeDtypeStruct((PAD_ROWS, D_MODEL), jnp.float32),
            jax.ShapeDtypeStruct((N_TOKENS,), jnp.int32),
        ],
        scratch_types=[
            pltpu.VMEM((TOK_PER_W,), jnp.int32),
            pltpu.VMEM((TOK_PER_W,), jnp.int32),
            pltpu.VMEM((L,), jnp.int32),
            pltpu.VMEM((TOK_PER_W,), jnp.int32),
            pltpu.VMEM((L, D_MODEL), jnp.float32),
            pltpu.SemaphoreType.DMA,
        ],
        compiler_params=pltpu.CompilerParams(needs_layout_passes=False),
    )
    def _dispatch(x_hbm, top1_hbm, pos_hbm, starts_hbm, xp_hbm, dest_hbm,
                  t_v, p_v, s_v, d_v, rows_v, sem):
        wid = lax.axis_index("s") * 2 + lax.axis_index("c")
        base = wid * TOK_PER_W
        pltpu.sync_copy(top1_hbm.at[pl.ds(base, TOK_PER_W)], t_v)
        pltpu.sync_copy(pos_hbm.at[pl.ds(base, TOK_PER_W)], p_v)
        pltpu.sync_copy(starts_hbm, s_v)
        for j in range(CH):
            e = t_v[pl.ds(j * L, L)]
            s = plsc.load_gather(s_v, [e])
            d_v[pl.ds(j * L, L)] = s + p_v[pl.ds(j * L, L)]
        pltpu.sync_copy(d_v, dest_hbm.at[pl.ds(base, TOK_PER_W)])
        for j in range(CH):
            pltpu.sync_copy(x_hbm.at[pl.ds(base + j * L, L)], rows_v)
            d = d_v[pl.ds(j * L, L)]
            pltpu.async_copy(rows_v, xp_hbm.at[d], sem).wait()

    @functools.partial(
        pl.kernel,
        mesh=mesh,
        out_type=jax.ShapeDtypeStruct((N_TOKENS, D_MODEL), jnp.float32),
        scratch_types=[
            pltpu.VMEM((TOK_PER_W,), jnp.int32),
            pltpu.VMEM((L, D_MODEL), jnp.float32),
            pltpu.SemaphoreType.DMA,
        ],
        compiler_params=pltpu.CompilerParams(needs_layout_passes=False),
    )
    def _combine(op_hbm, dest_hbm, out_hbm, d_v, rows_v, sem):
        wid = lax.axis_index("s") * 2 + lax.axis_index("c")
        base = wid * TOK_PER_W
        pltpu.sync_copy(dest_hbm.at[pl.ds(base, TOK_PER_W)], d_v)
        for j in range(CH):
            d = d_v[pl.ds(j * L, L)]
            pltpu.async_copy(op_hbm.at[d], rows_v, sem).wait()
            pltpu.sync_copy(rows_v, out_hbm.at[pl.ds(base + j * L, L)])

    return _dispatch, _combine


F_CHUNKS = 4
F_CHUNK = D_FF // F_CHUNKS


def _ffn_body(te_ref, na_ref, x_ref, w1_ref, b1_ref, w2_ref, b2_ref, out_ref,
              acc_ref):
    c = pl.program_id(0)
    i = pl.program_id(1)

    @pl.when(i < na_ref[0])
    def _():
        x = x_ref[...]
        h = lax.dot_general(x, w1_ref[0], (((1,), (1,)), ((), ())),
                            preferred_element_type=jnp.float32)
        h = h + b1_ref[0]
        h = h * jax.nn.sigmoid(h)
        part = lax.dot_general(h, w2_ref[0], (((1,), (1,)), ((), ())),
                               preferred_element_type=jnp.float32)
        rows = pl.ds(i * T, T)

        @pl.when(c == 0)
        def _():
            acc_ref[rows, :] = part + b2_ref[0]

        @pl.when((c > 0) & (c < F_CHUNKS - 1))
        def _():
            acc_ref[rows, :] += part

        @pl.when(c == F_CHUNKS - 1)
        def _():
            out_ref[...] = acc_ref[rows, :] + part


_ffn_call = pl.pallas_call(
    _ffn_body,
    grid_spec=pltpu.PrefetchScalarGridSpec(
        num_scalar_prefetch=2,
        grid=(F_CHUNKS, MAX_TILES),
        in_specs=[
            pl.BlockSpec((T, D_MODEL),
                         lambda c, i, te, na: (jnp.where(i < na[0], i, 0), 0)),
            pl.BlockSpec((1, F_CHUNK, D_MODEL),
                         lambda c, i, te, na: (te[i], c, 0)),
            pl.BlockSpec((1, 1, F_CHUNK),
                         lambda c, i, te, na: (te[i], 0, c)),
            pl.BlockSpec((1, D_MODEL, F_CHUNK),
                         lambda c, i, te, na: (te[i], 0, c)),
            pl.BlockSpec((1, 1, D_MODEL),
                         lambda c, i, te, na: (te[i], 0, 0)),
        ],
        out_specs=pl.BlockSpec(
            (T, D_MODEL),
            lambda c, i, te, na: (
                jnp.where(c == F_CHUNKS - 1,
                          jnp.minimum(i, na[0] - 1), na[0] - 1), 0)),
        scratch_shapes=[pltpu.VMEM((PAD_ROWS, D_MODEL), jnp.float32)],
    ),
    out_shape=jax.ShapeDtypeStruct((PAD_ROWS, D_MODEL), jnp.float32),
    compiler_params=pltpu.CompilerParams(
        dimension_semantics=("arbitrary", "arbitrary"),
        vmem_limit_bytes=64 * 1024 * 1024,
    ),
)


def kernel(x, gate_W, gate_b, W1, b1, W2, b2):
    b, s, d = x.shape
    x_flat = x.reshape(b * s, d)

    top1, pos, cnt128, loss = _gate_call(x_flat, gate_W,
                                         gate_b.reshape(1, N_EXP))
    counts = cnt128[0, :N_EXP]

    tiles_per = (counts + (T - 1)) // T
    tile_bounds = jnp.cumsum(tiles_per)
    n_active = tile_bounds[-1]
    starts = (tile_bounds - tiles_per) * T
    tile_ids = jnp.arange(MAX_TILES, dtype=jnp.int32)
    te = jnp.sum((tile_ids[:, None] >= tile_bounds[None, :]).astype(jnp.int32),
                 axis=1)
    te_last = jnp.take(te, n_active - 1)
    te = jnp.where(tile_ids < n_active, te, te_last).astype(jnp.int32)
    starts_pad = jnp.pad(starts, (0, L - N_EXP)).astype(jnp.int32)

    _dispatch, _combine = _sc_kernels()
    x_padded, dest = _dispatch(x_flat, top1.reshape(-1), pos.reshape(-1),
                               starts_pad)
    out_padded = _ffn_call(te, n_active.reshape(1).astype(jnp.int32),
                           x_padded, W1, b1.reshape(N_EXP, 1, D_FF), W2,
                           b2.reshape(N_EXP, 1, D_MODEL))
    out_flat = _combine(out_padded, dest)
    return out_flat.reshape(b, s, d), loss[0, 0]


# speedup vs baseline: 1.1986x; 1.1986x over previous
import functools

import jax
import jax.numpy as jnp
from jax import lax
from jax.experimental import pallas as pl
from jax.experimental.pallas import tpu as pltpu
from jax.experimental.pallas import tpu_sc as plsc

D_MODEL = 1024
D_FF = 4096
N_EXP = 8
N_TOKENS = 4096
GATE_TILE = 1024
T = 512
MAX_TILES = N_TOKENS // T + N_EXP
PAD_ROWS = MAX_TILES * T

NW = 32
TOK_PER_W = N_TOKENS // NW
L = 16
CH = TOK_PER_W // L


def _gate_body(x_ref, gw_ref, gb_ref, top1_ref, pos_ref, cnt_ref, loss_ref,
               carry_ref, imp_ref):
    i = pl.program_id(0)

    @pl.when(i == 0)
    def _():
        carry_ref[...] = jnp.zeros_like(carry_ref)
        imp_ref[...] = jnp.zeros_like(imp_ref)

    x = x_ref[...]
    logits = lax.dot_general(x, gw_ref[...], (((1,), (1,)), ((), ())),
                             preferred_element_type=jnp.float32)
    logits = logits + gb_ref[...]
    m = jnp.max(logits, axis=-1, keepdims=True)
    e = jnp.exp(logits - m)
    probs = e / jnp.sum(e, axis=-1, keepdims=True)
    top1 = jnp.argmax(probs, axis=-1).astype(jnp.int32)

    lanes = lax.broadcasted_iota(jnp.int32, (GATE_TILE, 128), 1)
    oh = (lanes == top1[:, None]).astype(jnp.float32)

    r = lax.broadcasted_iota(jnp.int32, (GATE_TILE, GATE_TILE), 0)
    c = lax.broadcasted_iota(jnp.int32, (GATE_TILE, GATE_TILE), 1)
    tri = (c < r).astype(jnp.bfloat16)
    excl = lax.dot_general(tri, oh.astype(jnp.bfloat16),
                           (((1,), (0,)), ((), ())),
                           preferred_element_type=jnp.float32)
    pos = jnp.sum((excl + carry_ref[...]) * oh, axis=1)

    top1_ref[...] = top1[:, None]
    pos_ref[...] = pos.astype(jnp.int32)[:, None]

    imp_ref[...] += jnp.sum(probs, axis=0, keepdims=True)
    new_carry = carry_ref[...] + jnp.sum(oh, axis=0, keepdims=True)
    carry_ref[...] = new_carry
    cnt_ref[...] = new_carry.astype(jnp.int32)
    imp_mean = imp_ref[...] / N_TOKENS
    load = new_carry[:, :N_EXP] / N_TOKENS
    loss_ref[...] = jnp.sum(N_EXP * imp_mean * load).reshape(1, 1)


_gate_call = pl.pallas_call(
    _gate_body,
    grid=(N_TOKENS // GATE_TILE,),
    in_specs=[
        pl.BlockSpec((GATE_TILE, D_MODEL), lambda i: (i, 0)),
        pl.BlockSpec((N_EXP, D_MODEL), lambda i: (0, 0)),
        pl.BlockSpec((1, N_EXP), lambda i: (0, 0)),
    ],
    out_specs=[
        pl.BlockSpec((GATE_TILE, 1), lambda i: (i, 0)),
        pl.BlockSpec((GATE_TILE, 1), lambda i: (i, 0)),
        pl.BlockSpec((1, 128), lambda i: (0, 0)),
        pl.BlockSpec((1, 1), lambda i: (0, 0)),
    ],
    out_shape=[
        jax.ShapeDtypeStruct((N_TOKENS, 1), jnp.int32),
        jax.ShapeDtypeStruct((N_TOKENS, 1), jnp.int32),
        jax.ShapeDtypeStruct((1, 128), jnp.int32),
        jax.ShapeDtypeStruct((1, 1), jnp.float32),
    ],
    scratch_shapes=[
        pltpu.VMEM((1, 128), jnp.float32),
        pltpu.VMEM((1, N_EXP), jnp.float32),
    ],
    compiler_params=pltpu.CompilerParams(
        dimension_semantics=("arbitrary",)),
)


@functools.lru_cache(maxsize=None)
def _sc_kernels():
    mesh = plsc.VectorSubcoreMesh(core_axis_name="c", subcore_axis_name="s")

    @functools.partial(
        pl.kernel,
        mesh=mesh,
        out_type=[
            jax.ShapeDtypeStruct((PAD_ROWS, D_MODEL), jnp.float32),
            jax.ShapeDtypeStruct((N_TOKENS,), jnp.int32),
            jax.ShapeDtypeStruct((L,), jnp.int32),
            jax.ShapeDtypeStruct((L,), jnp.int32),
        ],
        scratch_types=[
            pltpu.VMEM((TOK_PER_W,), jnp.int32),
            pltpu.VMEM((TOK_PER_W,), jnp.int32),
            pltpu.VMEM((L,), jnp.int32),
            pltpu.VMEM((L,), jnp.int32),
            pltpu.VMEM((L,), jnp.int32),
            pltpu.VMEM((L,), jnp.int32),
            pltpu.VMEM((TOK_PER_W,), jnp.int32),
            pltpu.VMEM((2, L, D_MODEL), jnp.float32),
            pltpu.SemaphoreType.DMA,
            pltpu.SemaphoreType.DMA,
            pltpu.SemaphoreType.DMA,
            pltpu.SemaphoreType.DMA,
        ],
        compiler_params=pltpu.CompilerParams(needs_layout_passes=False),
    )
    def _dispatch(x_hbm, top1_hbm, pos_hbm, cnt_hbm, xp_hbm, dest_hbm,
                  te_hbm, na_hbm, t_v, p_v, c_v, b_v, s_v, m_v, d_v, rows_v,
                  g0, g1, s0, s1):
        wid = lax.axis_index("s") * 2 + lax.axis_index("c")
        base = wid * TOK_PER_W
        pltpu.sync_copy(top1_hbm.at[pl.ds(base, TOK_PER_W)], t_v)
        pltpu.sync_copy(pos_hbm.at[pl.ds(base, TOK_PER_W)], p_v)
        pltpu.sync_copy(cnt_hbm, c_v)

        cv = c_v[...]
        tiles = (cv + (T - 1)) // T
        bounds = plsc.cumsum(tiles)
        na = jnp.max(bounds)
        b_v[...] = bounds
        s_v[...] = (bounds - tiles) * T
        iota = lax.iota(jnp.int32, L)
        te = jnp.zeros((L,), jnp.int32)
        for e in range(N_EXP):
            be = plsc.load_gather(b_v, [jnp.full((L,), e, jnp.int32)])
            te = te + (be <= iota).astype(jnp.int32)
        te_last = jnp.sum((tiles > 0).astype(jnp.int32)) - 1
        te = jnp.where(iota < na, te, te_last)

        @pl.when(wid == 0)
        def _():
            m_v[...] = te
            pltpu.sync_copy(m_v, te_hbm)
            m_v[...] = jnp.full((L,), na, jnp.int32)
            pltpu.sync_copy(m_v, na_hbm)

        for j in range(CH):
            e = t_v[pl.ds(j * L, L)]
            s = plsc.load_gather(s_v, [e])
            d_v[pl.ds(j * L, L)] = s + p_v[pl.ds(j * L, L)]
        pltpu.sync_copy(d_v, dest_hbm.at[pl.ds(base, TOK_PER_W)])

        def gath(j, b, sem):
            return pltpu.async_copy(
                x_hbm.at[pl.ds(base + j * L, L)], rows_v.at[b], sem)

        gh = [gath(0, 0, g0), gath(1, 1, g1)]
        gsem = [g0, g1]
        ssem = [s0, s1]
        last_sc = [None, None]
        for j in range(CH):
            b = j & 1
            gh[b].wait()
            d = d_v[pl.ds(j * L, L)]
            sc = pltpu.async_copy(rows_v.at[b], xp_hbm.at[d], ssem[b])
            last_sc[b] = sc
            if j + 2 < CH:
                sc.wait()
                last_sc[b] = None
                gh[b] = gath(j + 2, b, gsem[b])
        for sc in last_sc:
            if sc is not None:
                sc.wait()

    @functools.partial(
        pl.kernel,
        mesh=mesh,
        out_type=jax.ShapeDtypeStruct((N_TOKENS, D_MODEL), jnp.float32),
        scratch_types=[
            pltpu.VMEM((TOK_PER_W,), jnp.int32),
            pltpu.VMEM((2, L, D_MODEL), jnp.float32),
            pltpu.SemaphoreType.DMA,
            pltpu.SemaphoreType.DMA,
            pltpu.SemaphoreType.DMA,
            pltpu.SemaphoreType.DMA,
        ],
        compiler_params=pltpu.CompilerParams(needs_layout_passes=False),
    )
    def _combine(op_hbm, dest_hbm, out_hbm, d_v, rows_v, g0, g1, s0, s1):
        wid = lax.axis_index("s") * 2 + lax.axis_index("c")
        base = wid * TOK_PER_W
        pltpu.sync_copy(dest_hbm.at[pl.ds(base, TOK_PER_W)], d_v)

        def gath(j, b, sem):
            d = d_v[pl.ds(j * L, L)]
            return pltpu.async_copy(op_hbm.at[d], rows_v.at[b], sem)

        gh = [gath(0, 0, g0), gath(1, 1, g1)]
        gsem = [g0, g1]
        ssem = [s0, s1]
        last_sc = [None, None]
        for j in range(CH):
            b = j & 1
            gh[b].wait()
            sc = pltpu.async_copy(
                rows_v.at[b], out_hbm.at[pl.ds(base + j * L, L)], ssem[b])
            last_sc[b] = sc
            if j + 2 < CH:
                sc.wait()
                last_sc[b] = None
                gh[b] = gath(j + 2, b, gsem[b])
        for sc in last_sc:
            if sc is not None:
                sc.wait()

    return _dispatch, _combine


F_CHUNKS = 2
F_CHUNK = D_FF // F_CHUNKS


def _ffn_body(te_ref, na_ref, x_ref, w1_ref, b1_ref, w2_ref, b2_ref, out_ref):
    i = pl.program_id(0)
    c = pl.program_id(1)

    @pl.when(i < na_ref[0])
    def _():
        x = x_ref[...]
        h = lax.dot_general(x, w1_ref[0], (((1,), (1,)), ((), ())),
                            preferred_element_type=jnp.float32)
        h = h + b1_ref[0]
        h = h * jax.nn.sigmoid(h)
        part = lax.dot_general(h, w2_ref[0], (((1,), (1,)), ((), ())),
                               preferred_element_type=jnp.float32)

        @pl.when(c == 0)
        def _():
            out_ref[...] = part + b2_ref[0]

        @pl.when(c != 0)
        def _():
            out_ref[...] += part


def _clamp_c(i, c, na):
    return jnp.where(i < na[0], c, F_CHUNKS - 1)


_ffn_call = pl.pallas_call(
    _ffn_body,
    grid_spec=pltpu.PrefetchScalarGridSpec(
        num_scalar_prefetch=2,
        grid=(MAX_TILES, F_CHUNKS),
        in_specs=[
            pl.BlockSpec((T, D_MODEL),
                         lambda i, c, te, na: (jnp.where(i < na[0], i, 0), 0)),
            pl.BlockSpec((1, F_CHUNK, D_MODEL),
                         lambda i, c, te, na: (te[i], _clamp_c(i, c, na), 0)),
            pl.BlockSpec((1, 1, F_CHUNK),
                         lambda i, c, te, na: (te[i], 0, _clamp_c(i, c, na))),
            pl.BlockSpec((1, D_MODEL, F_CHUNK),
                         lambda i, c, te, na: (te[i], 0, _clamp_c(i, c, na))),
            pl.BlockSpec((1, 1, D_MODEL),
                         lambda i, c, te, na: (te[i], 0, 0)),
        ],
        out_specs=pl.BlockSpec(
            (T, D_MODEL),
            lambda i, c, te, na: (jnp.minimum(i, na[0] - 1), 0)),
    ),
    out_shape=jax.ShapeDtypeStruct((PAD_ROWS, D_MODEL), jnp.float32),
    compiler_params=pltpu.CompilerParams(
        dimension_semantics=("arbitrary", "arbitrary"),
        vmem_limit_bytes=64 * 1024 * 1024,
    ),
)


def kernel(x, gate_W, gate_b, W1, b1, W2, b2):
    b, s, d = x.shape
    x_flat = x.reshape(b * s, d)

    top1, pos, cnt128, loss = _gate_call(x_flat, gate_W,
                                         gate_b.reshape(1, N_EXP))
    cnt16 = cnt128[0, :L]

    _dispatch, _combine = _sc_kernels()
    x_padded, dest, te, na = _dispatch(x_flat, top1.reshape(-1),
                                       pos.reshape(-1), cnt16)
    out_padded = _ffn_call(te, na, x_padded, W1,
                           b1.reshape(N_EXP, 1, D_FF), W2,
                           b2.reshape(N_EXP, 1, D_MODEL))
    out_flat = _combine(out_padded, dest)
    return out_flat.reshape(b, s, d), loss[0, 0]


# speedup vs baseline: 1.2037x; 1.0042x over previous
import functools

import jax
import jax.numpy as jnp
from jax import lax
from jax.experimental import pallas as pl
from jax.experimental.pallas import tpu as pltpu
from jax.experimental.pallas import tpu_sc as plsc

D_MODEL = 1024
D_FF = 4096
N_EXP = 8
N_TOKENS = 4096
GATE_TILE = 1024
T = 512
MAX_TILES = N_TOKENS // T + N_EXP
PAD_ROWS = MAX_TILES * T

NW = 32
TOK_PER_W = N_TOKENS // NW
L = 16
CH = TOK_PER_W // L


def _gate_body(x_ref, gw_ref, gb_ref, top1_ref, pos_ref, cnt_ref, loss_ref,
               carry_ref, imp_ref):
    i = pl.program_id(0)

    @pl.when(i == 0)
    def _():
        carry_ref[...] = jnp.zeros_like(carry_ref)
        imp_ref[...] = jnp.zeros_like(imp_ref)

    x = x_ref[...]
    logits = lax.dot_general(x, gw_ref[...], (((1,), (1,)), ((), ())),
                             preferred_element_type=jnp.float32)
    logits = logits + gb_ref[...]
    m = jnp.max(logits, axis=-1, keepdims=True)
    e = jnp.exp(logits - m)
    probs = e / jnp.sum(e, axis=-1, keepdims=True)
    top1 = jnp.argmax(probs, axis=-1).astype(jnp.int32)

    lanes = lax.broadcasted_iota(jnp.int32, (GATE_TILE, 128), 1)
    oh = (lanes == top1[:, None]).astype(jnp.float32)

    r = lax.broadcasted_iota(jnp.int32, (GATE_TILE, GATE_TILE), 0)
    c = lax.broadcasted_iota(jnp.int32, (GATE_TILE, GATE_TILE), 1)
    tri = (c < r).astype(jnp.bfloat16)
    excl = lax.dot_general(tri, oh.astype(jnp.bfloat16),
                           (((1,), (0,)), ((), ())),
                           preferred_element_type=jnp.float32)
    pos = jnp.sum((excl + carry_ref[...]) * oh, axis=1)

    top1_ref[...] = top1[:, None]
    pos_ref[...] = pos.astype(jnp.int32)[:, None]

    imp_ref[...] += jnp.sum(probs, axis=0, keepdims=True)
    new_carry = carry_ref[...] + jnp.sum(oh, axis=0, keepdims=True)
    carry_ref[...] = new_carry
    cnt_ref[...] = new_carry.astype(jnp.int32)
    imp_mean = imp_ref[...] / N_TOKENS
    load = new_carry[:, :N_EXP] / N_TOKENS
    loss_ref[...] = jnp.sum(N_EXP * imp_mean * load).reshape(1, 1)


_gate_call = pl.pallas_call(
    _gate_body,
    grid=(N_TOKENS // GATE_TILE,),
    in_specs=[
        pl.BlockSpec((GATE_TILE, D_MODEL), lambda i: (i, 0)),
        pl.BlockSpec((N_EXP, D_MODEL), lambda i: (0, 0)),
        pl.BlockSpec((1, N_EXP), lambda i: (0, 0)),
    ],
    out_specs=[
        pl.BlockSpec((GATE_TILE, 1), lambda i: (i, 0)),
        pl.BlockSpec((GATE_TILE, 1), lambda i: (i, 0)),
        pl.BlockSpec((1, 128), lambda i: (0, 0)),
        pl.BlockSpec((1, 1), lambda i: (0, 0)),
    ],
    out_shape=[
        jax.ShapeDtypeStruct((N_TOKENS, 1), jnp.int32),
        jax.ShapeDtypeStruct((N_TOKENS, 1), jnp.int32),
        jax.ShapeDtypeStruct((1, 128), jnp.int32),
        jax.ShapeDtypeStruct((1, 1), jnp.float32),
    ],
    scratch_shapes=[
        pltpu.VMEM((1, 128), jnp.float32),
        pltpu.VMEM((1, N_EXP), jnp.float32),
    ],
    compiler_params=pltpu.CompilerParams(
        dimension_semantics=("arbitrary",)),
)


@functools.lru_cache(maxsize=None)
def _sc_kernels():
    mesh = plsc.VectorSubcoreMesh(core_axis_name="c", subcore_axis_name="s")

    @functools.partial(
        pl.kernel,
        mesh=mesh,
        out_type=[
            jax.ShapeDtypeStruct((PAD_ROWS, D_MODEL), jnp.float32),
            jax.ShapeDtypeStruct((N_TOKENS,), jnp.int32),
            jax.ShapeDtypeStruct((L,), jnp.int32),
            jax.ShapeDtypeStruct((L,), jnp.int32),
        ],
        scratch_types=[
            pltpu.VMEM((TOK_PER_W,), jnp.int32),
            pltpu.VMEM((TOK_PER_W,), jnp.int32),
            pltpu.VMEM((L,), jnp.int32),
            pltpu.VMEM((L,), jnp.int32),
            pltpu.VMEM((L,), jnp.int32),
            pltpu.VMEM((L,), jnp.int32),
            pltpu.VMEM((TOK_PER_W,), jnp.int32),
            pltpu.VMEM((2, L, D_MODEL), jnp.float32),
            pltpu.SemaphoreType.DMA,
            pltpu.SemaphoreType.DMA,
            pltpu.SemaphoreType.DMA,
            pltpu.SemaphoreType.DMA,
        ],
        compiler_params=pltpu.CompilerParams(needs_layout_passes=False),
    )
    def _dispatch(x_hbm, top1_hbm, pos_hbm, cnt_hbm, xp_hbm, dest_hbm,
                  te_hbm, na_hbm, t_v, p_v, c_v, b_v, s_v, m_v, d_v, rows_v,
                  g0, g1, s0, s1):
        wid = lax.axis_index("s") * 2 + lax.axis_index("c")
        base = wid * TOK_PER_W
        pltpu.sync_copy(top1_hbm.at[pl.ds(base, TOK_PER_W)], t_v)
        pltpu.sync_copy(pos_hbm.at[pl.ds(base, TOK_PER_W)], p_v)
        pltpu.sync_copy(cnt_hbm, c_v)

        cv = c_v[...]
        tiles = (cv + (T - 1)) // T
        bounds = plsc.cumsum(tiles)
        iota = lax.iota(jnp.int32, L)
        na = jnp.sum(jnp.where(iota == N_EXP - 1, bounds, 0))
        b_v[...] = bounds
        s_v[...] = (bounds - tiles) * T
        te = jnp.zeros((L,), jnp.int32)
        for e in range(N_EXP):
            be = jnp.sum(jnp.where(iota == e, bounds, 0))
            te = te + (be <= iota).astype(jnp.int32)
        te_last = jnp.sum((tiles > 0).astype(jnp.int32)) - 1
        te = jnp.where(iota < na, te, te_last)

        @pl.when(wid == 0)
        def _():
            m_v[...] = te
            pltpu.sync_copy(m_v, te_hbm)
            m_v[...] = jnp.full((L,), na, jnp.int32)
            pltpu.sync_copy(m_v, na_hbm)

        for j in range(CH):
            e = t_v[pl.ds(j * L, L)]
            s = plsc.load_gather(s_v, [e])
            d_v[pl.ds(j * L, L)] = s + p_v[pl.ds(j * L, L)]
        pltpu.sync_copy(d_v, dest_hbm.at[pl.ds(base, TOK_PER_W)])

        def gath(j, b, sem):
            return pltpu.async_copy(
                x_hbm.at[pl.ds(base + j * L, L)], rows_v.at[b], sem)

        gh = [gath(0, 0, g0), gath(1, 1, g1)]
        gsem = [g0, g1]
        ssem = [s0, s1]
        last_sc = [None, None]
        for j in range(CH):
            b = j & 1
            gh[b].wait()
            d = d_v[pl.ds(j * L, L)]
            sc = pltpu.async_copy(rows_v.at[b], xp_hbm.at[d], ssem[b])
            last_sc[b] = sc
            if j + 2 < CH:
                sc.wait()
                last_sc[b] = None
                gh[b] = gath(j + 2, b, gsem[b])
        for sc in last_sc:
            if sc is not None:
                sc.wait()

    @functools.partial(
        pl.kernel,
        mesh=mesh,
        out_type=jax.ShapeDtypeStruct((N_TOKENS, D_MODEL), jnp.float32),
        scratch_types=[
            pltpu.VMEM((TOK_PER_W,), jnp.int32),
            pltpu.VMEM((2, L, D_MODEL), jnp.float32),
            pltpu.SemaphoreType.DMA,
            pltpu.SemaphoreType.DMA,
            pltpu.SemaphoreType.DMA,
            pltpu.SemaphoreType.DMA,
        ],
        compiler_params=pltpu.CompilerParams(needs_layout_passes=False),
    )
    def _combine(op_hbm, dest_hbm, out_hbm, d_v, rows_v, g0, g1, s0, s1):
        wid = lax.axis_index("s") * 2 + lax.axis_index("c")
        base = wid * TOK_PER_W
        pltpu.sync_copy(dest_hbm.at[pl.ds(base, TOK_PER_W)], d_v)

        def gath(j, b, sem):
            d = d_v[pl.ds(j * L, L)]
            return pltpu.async_copy(op_hbm.at[d], rows_v.at[b], sem)

        gh = [gath(0, 0, g0), gath(1, 1, g1)]
        gsem = [g0, g1]
        ssem = [s0, s1]
        last_sc = [None, None]
        for j in range(CH):
            b = j & 1
            gh[b].wait()
            sc = pltpu.async_copy(
                rows_v.at[b], out_hbm.at[pl.ds(base + j * L, L)], ssem[b])
            last_sc[b] = sc
            if j + 2 < CH:
                sc.wait()
                last_sc[b] = None
                gh[b] = gath(j + 2, b, gsem[b])
        for sc in last_sc:
            if sc is not None:
                sc.wait()

    return _dispatch, _combine


F_CHUNKS = 2
F_CHUNK = D_FF // F_CHUNKS


def _ffn_body(te_ref, na_ref, x_ref, w1_ref, b1_ref, w2_ref, b2_ref, out_ref):
    i = pl.program_id(0)
    c = pl.program_id(1)

    @pl.when(i < na_ref[0])
    def _():
        x = x_ref[...]
        h = lax.dot_general(x, w1_ref[0], (((1,), (1,)), ((), ())),
                            preferred_element_type=jnp.float32)
        h = h + b1_ref[0]
        h = h * jax.nn.sigmoid(h)
        part = lax.dot_general(h, w2_ref[0], (((1,), (1,)), ((), ())),
                               preferred_element_type=jnp.float32)

        @pl.when(c == 0)
        def _():
            out_ref[...] = part + b2_ref[0]

        @pl.when(c != 0)
        def _():
            out_ref[...] += part


def _clamp_c(i, c, na):
    return jnp.where(i < na[0], c, F_CHUNKS - 1)


_ffn_call = pl.pallas_call(
    _ffn_body,
    grid_spec=pltpu.PrefetchScalarGridSpec(
        num_scalar_prefetch=2,
        grid=(MAX_TILES, F_CHUNKS),
        in_specs=[
            pl.BlockSpec((T, D_MODEL),
                         lambda i, c, te, na: (jnp.where(i < na[0], i, 0), 0)),
            pl.BlockSpec((1, F_CHUNK, D_MODEL),
                         lambda i, c, te, na: (te[i], _clamp_c(i, c, na), 0)),
            pl.BlockSpec((1, 1, F_CHUNK),
                         lambda i, c, te, na: (te[i], 0, _clamp_c(i, c, na))),
            pl.BlockSpec((1, D_MODEL, F_CHUNK),
                         lambda i, c, te, na: (te[i], 0, _clamp_c(i, c, na))),
            pl.BlockSpec((1, 1, D_MODEL),
                         lambda i, c, te, na: (te[i], 0, 0)),
        ],
        out_specs=pl.BlockSpec(
            (T, D_MODEL),
            lambda i, c, te, na: (jnp.minimum(i, na[0] - 1), 0)),
    ),
    out_shape=jax.ShapeDtypeStruct((PAD_ROWS, D_MODEL), jnp.float32),
    compiler_params=pltpu.CompilerParams(
        dimension_semantics=("arbitrary", "arbitrary"),
        vmem_limit_bytes=64 * 1024 * 1024,
    ),
)


def kernel(x, gate_W, gate_b, W1, b1, W2, b2):
    b, s, d = x.shape
    x_flat = x.reshape(b * s, d)

    top1, pos, cnt128, loss = _gate_call(x_flat, gate_W,
                                         gate_b.reshape(1, N_EXP))
    cnt16 = cnt128[0, :L]

    _dispatch, _combine = _sc_kernels()
    x_padded, dest, te, na = _dispatch(x_flat, top1.reshape(-1),
                                       pos.reshape(-1), cnt16)
    out_padded = _ffn_call(te, na, x_padded, W1,
                           b1.reshape(N_EXP, 1, D_FF), W2,
                           b2.reshape(N_EXP, 1, D_MODEL))
    out_flat = _combine(out_padded, dest)
    return out_flat.reshape(b, s, d), loss[0, 0]
